# Initial kernel scaffold; baseline (speedup 1.0000x reference)
import jax
import jax.numpy as jnp
from jax.experimental import pallas as pl


def kernel(H, E, heads, queries, W, b):
    raise NotImplementedError("write your pallas kernel here")


# speedup vs baseline: 2.6399x; 2.6399x over previous
import functools

import jax
import jax.numpy as jnp
from jax import lax
from jax.experimental import pallas as pl
from jax.experimental.pallas import tpu as pltpu
from jax.experimental.pallas import tpu_sc as plsc

N_NODES = 10000
N_EDGES = 320000
D = 128

NC = 2
NS = 16
NW = NC * NS
EDGES_PER_W = N_EDGES // NW
CHUNK = 400
N_CHUNKS = EDGES_PER_W // CHUNK


def _mm_bias_kernel(h_ref, w1t_ref, b_ref, o_ref):
    o_ref[...] = (
        jnp.dot(h_ref[...], w1t_ref[...], preferred_element_type=jnp.float32)
        + b_ref[...]
    )


def _node_transform(H, W1t, b2d):
    return pl.pallas_call(
        _mm_bias_kernel,
        out_shape=jax.ShapeDtypeStruct((N_NODES, D), jnp.float32),
    )(H, W1t, b2d)


def _sc_gather_body(table_hbm, idx_hbm, out_hbm, idx_v, rows_v, sem):
    wid = lax.axis_index("s") * NC + lax.axis_index("c")
    base = wid * EDGES_PER_W

    def body(i, carry):
        off = base + i * CHUNK
        pltpu.sync_copy(idx_hbm.at[pl.ds(off, CHUNK)], idx_v)
        pltpu.async_copy(table_hbm.at[idx_v], rows_v, sem).wait()
        pltpu.sync_copy(rows_v, out_hbm.at[pl.ds(off, CHUNK)])
        return carry

    lax.fori_loop(0, N_CHUNKS, body, 0)


def _sc_gather(table, heads):
    mesh = plsc.VectorSubcoreMesh(core_axis_name="c", subcore_axis_name="s")
    k = functools.partial(
        pl.kernel,
        mesh=mesh,
        out_type=jax.ShapeDtypeStruct((N_EDGES, D), jnp.float32),
        scratch_types=[
            pltpu.VMEM((CHUNK,), jnp.int32),
            pltpu.VMEM((CHUNK, D), jnp.float32),
            pltpu.SemaphoreType.DMA,
        ],
    )(_sc_gather_body)
    return k(table, heads)


def _add_mm_kernel(g_ref, e_ref, w2t_ref, o_ref):
    o_ref[...] = g_ref[...] + jnp.dot(
        e_ref[...], w2t_ref[...], preferred_element_type=jnp.float32
    )


def _edge_transform(G, E, W2t, blk):
    n_blocks = N_EDGES // blk
    return pl.pallas_call(
        _add_mm_kernel,
        grid=(n_blocks,),
        in_specs=[
            pl.BlockSpec((blk, D), lambda i: (i, 0)),
            pl.BlockSpec((blk, D), lambda i: (i, 0)),
            pl.BlockSpec((D, D), lambda i: (0, 0)),
        ],
        out_specs=pl.BlockSpec((blk, D), lambda i: (i, 0)),
        out_shape=jax.ShapeDtypeStruct((N_EDGES, D), jnp.float32),
    )(G, E, W2t)


@jax.jit
def kernel(H, E, heads, queries, W, b):
    W1t = W[:, :D].T
    W2t = W[:, D:].T
    b2d = b.reshape(1, D)
    T = _node_transform(H, W1t, b2d)
    G = _sc_gather(T, heads.astype(jnp.int32))
    return _edge_transform(G, E, W2t, blk=2000)


# speedup vs baseline: 2.9874x; 1.1316x over previous
import functools

import jax
import jax.numpy as jnp
from jax import lax
from jax.experimental import pallas as pl
from jax.experimental.pallas import tpu as pltpu
from jax.experimental.pallas import tpu_sc as plsc

N_NODES = 10000
N_EDGES = 320000
D = 128

NC = 2
NS = 16
NW = NC * NS
EDGES_PER_W = N_EDGES // NW
CHUNK = 400
N_CHUNKS = EDGES_PER_W // CHUNK


def _mm_bias_kernel(h_ref, w1t_ref, b_ref, o_ref):
    o_ref[...] = (
        jnp.dot(h_ref[...], w1t_ref[...], preferred_element_type=jnp.float32)
        + b_ref[...]
    )


def _node_transform(H, W1t, b2d):
    return pl.pallas_call(
        _mm_bias_kernel,
        out_shape=jax.ShapeDtypeStruct((N_NODES, D), jnp.float32),
    )(H, W1t, b2d)


def _sc_gather_body(table_hbm, idx_hbm, out_hbm, idx_v, rows_v, sem):
    wid = lax.axis_index("s") * NC + lax.axis_index("c")
    base = wid * EDGES_PER_W

    def body(i, carry):
        off = base + i * CHUNK
        pltpu.sync_copy(idx_hbm.at[pl.ds(off, CHUNK)], idx_v)
        pltpu.async_copy(table_hbm.at[idx_v], rows_v, sem).wait()
        pltpu.sync_copy(rows_v, out_hbm.at[pl.ds(off, CHUNK)])
        return carry

    lax.fori_loop(0, N_CHUNKS, body, 0)


def _sc_gather(table, heads):
    mesh = plsc.VectorSubcoreMesh(core_axis_name="c", subcore_axis_name="s")
    k = functools.partial(
        pl.kernel,
        mesh=mesh,
        out_type=jax.ShapeDtypeStruct((N_EDGES, D), jnp.float32),
        scratch_types=[
            pltpu.VMEM((CHUNK,), jnp.int32),
            pltpu.VMEM((CHUNK, D), jnp.float32),
            pltpu.SemaphoreType.DMA,
        ],
    )(_sc_gather_body)
    return k(table, heads)


def _add_mm_kernel(g_ref, e_ref, w2t_ref, o_ref):
    o_ref[...] = g_ref[...] + jnp.dot(
        e_ref[...], w2t_ref[...], preferred_element_type=jnp.float32
    )


def _edge_transform(G, E, W2t, blk):
    n_blocks = N_EDGES // blk
    return pl.pallas_call(
        _add_mm_kernel,
        grid=(n_blocks,),
        in_specs=[
            pl.BlockSpec((blk, D), lambda i: (i, 0)),
            pl.BlockSpec((blk, D), lambda i: (i, 0)),
            pl.BlockSpec((D, D), lambda i: (0, 0)),
        ],
        out_specs=pl.BlockSpec((blk, D), lambda i: (i, 0)),
        out_shape=jax.ShapeDtypeStruct((N_EDGES, D), jnp.float32),
    )(G, E, W2t)


@jax.jit
def kernel(H, E, heads, queries, W, b):
    W1t = W[:, :D].T
    W2t = W[:, D:].T
    b2d = b.reshape(1, D)
    T = _node_transform(H, W1t, b2d)
    G = _sc_gather(T, heads.astype(jnp.int32))
    return _edge_transform(G, E, W2t, blk=4000)


# speedup vs baseline: 3.0466x; 1.0198x over previous
import functools

import jax
import jax.numpy as jnp
from jax import lax
from jax.experimental import pallas as pl
from jax.experimental.pallas import tpu as pltpu
from jax.experimental.pallas import tpu_sc as plsc

N_NODES = 10000
N_EDGES = 320000
D = 128

NC = 2
NS = 16
NW = NC * NS
EDGES_PER_W = N_EDGES // NW
CHUNK = 400
N_CHUNKS = EDGES_PER_W // CHUNK


def _mm_bias_kernel(h_ref, w1t_ref, b_ref, o_ref):
    o_ref[...] = (
        jnp.dot(h_ref[...], w1t_ref[...], preferred_element_type=jnp.float32)
        + b_ref[...]
    )


def _node_transform(H, W1t, b2d):
    return pl.pallas_call(
        _mm_bias_kernel,
        out_shape=jax.ShapeDtypeStruct((N_NODES, D), jnp.float32),
    )(H, W1t, b2d)


def _sc_gather_body(table_hbm, idx_hbm, out_hbm, idx_v, rows_v, sem):
    wid = lax.axis_index("s") * NC + lax.axis_index("c")
    base = wid * EDGES_PER_W

    def body(i, carry):
        off = base + i * CHUNK
        pltpu.sync_copy(idx_hbm.at[pl.ds(off, CHUNK)], idx_v)
        pltpu.async_copy(table_hbm.at[idx_v], rows_v, sem).wait()
        pltpu.sync_copy(rows_v, out_hbm.at[pl.ds(off, CHUNK)])
        return carry

    lax.fori_loop(0, N_CHUNKS, body, 0)


def _sc_gather(table, heads):
    mesh = plsc.VectorSubcoreMesh(core_axis_name="c", subcore_axis_name="s")
    k = functools.partial(
        pl.kernel,
        mesh=mesh,
        out_type=jax.ShapeDtypeStruct((N_EDGES, D), jnp.float32),
        scratch_types=[
            pltpu.VMEM((CHUNK,), jnp.int32),
            pltpu.VMEM((CHUNK, D), jnp.float32),
            pltpu.SemaphoreType.DMA,
        ],
    )(_sc_gather_body)
    return k(table, heads)


def _add_mm_kernel(g_ref, e_ref, w2t_ref, o_ref):
    o_ref[...] = g_ref[...] + jnp.dot(
        e_ref[...], w2t_ref[...], preferred_element_type=jnp.float32
    )


def _edge_transform(G, E, W2t, blk):
    n_blocks = N_EDGES // blk
    return pl.pallas_call(
        _add_mm_kernel,
        grid=(n_blocks,),
        in_specs=[
            pl.BlockSpec((blk, D), lambda i: (i, 0)),
            pl.BlockSpec((blk, D), lambda i: (i, 0)),
            pl.BlockSpec((D, D), lambda i: (0, 0)),
        ],
        out_specs=pl.BlockSpec((blk, D), lambda i: (i, 0)),
        out_shape=jax.ShapeDtypeStruct((N_EDGES, D), jnp.float32),
    )(G, E, W2t)


@jax.jit
def kernel(H, E, heads, queries, W, b):
    W1t = W[:, :D].T
    W2t = W[:, D:].T
    b2d = b.reshape(1, D)
    T = _node_transform(H, W1t, b2d)
    G = _sc_gather(T, heads.astype(jnp.int32))
    return _edge_transform(G, E, W2t, blk=8000)
